# baseline (speedup 1.0000x reference)
import numpy as np
import jax
import jax.numpy as jnp
from jax.experimental import pallas as pl
from jax.experimental.pallas import tpu as pltpu

_K = 25
_PAD = 12


def _band_weight(l: int) -> np.ndarray:
    w = np.zeros((l, l), np.float32)
    for t in range(l):
        for j in range(_K):
            src = min(max(t + j - _PAD, 0), l - 1)
            w[t, src] += 1.0 / _K
    return w


def _ma_body(w_ref, x_ref, o_ref):
    w = w_ref[...]
    for b in range(x_ref.shape[0]):
        xb = x_ref[b].astype(jnp.bfloat16)
        o_ref[b] = jnp.dot(w, xb, preferred_element_type=jnp.float32)


def kernel(x):
    b, l, c = x.shape
    w = jnp.asarray(_band_weight(l), dtype=jnp.bfloat16)
    b_blk = 1
    for cand in (8, 4, 2):
        if b % cand == 0 and b // cand >= 2:
            b_blk = cand
            break
    block_bytes = b_blk * l * c * x.dtype.itemsize
    vmem_limit = int(min(max(6 * block_bytes, 16 << 20), 64 << 20))
    return pl.pallas_call(
        _ma_body,
        out_shape=jax.ShapeDtypeStruct((b, l, c), x.dtype),
        grid=(b // b_blk,),
        in_specs=[
            pl.BlockSpec((l, l), lambda i: (0, 0)),
            pl.BlockSpec((b_blk, l, c), lambda i: (i, 0, 0)),
        ],
        out_specs=pl.BlockSpec((b_blk, l, c), lambda i: (i, 0, 0)),
        compiler_params=pltpu.CompilerParams(
            dimension_semantics=("parallel",),
            vmem_limit_bytes=vmem_limit,
        ),
    )(w, x)
